# baseline (speedup 1.0000x reference)
import functools

import jax
import jax.numpy as jnp
from jax import lax
from jax.experimental import pallas as pl
from jax.experimental.pallas import tpu as pltpu

_EPS = 1e-5
_INV_SQRT2 = 0.7071067811865476


def _layernorm(v, g, b):
    mu = jnp.mean(v, axis=-1, keepdims=True)
    c = v - mu
    var = jnp.mean(c * c, axis=-1, keepdims=True)
    return c * lax.rsqrt(var + _EPS) * g + b


def _erf_poly(z):
    c1, c2, c3, c4, c5 = (0.254829592, -0.284496736, 1.421413741,
                          -1.453152027, 1.061405429)
    az = jnp.abs(z)
    t = 1.0 / (1.0 + 0.3275911 * az)
    poly = ((((c5 * t + c4) * t + c3) * t + c2) * t + c1) * t
    y = 1.0 - poly * jnp.exp(-az * az)
    return jnp.where(z < 0.0, -y, y)


def _gelu(v):
    return 0.5 * v * (1.0 + _erf_poly(v * _INV_SQRT2))


def _dot(a, b):
    return jnp.dot(a, b, preferred_element_type=jnp.float32)


def _s2_stage_body(x_ref, ln1g, ln1b, w1a, w1b, w1c, b1a, b1b, b1c,
                   sa1, sa2a, sa2b, sa2c, w2, b2, ln2g, ln2b,
                   f1w, f1b, f2w, f2b, o_ref, *, H, W, lps):
    d = pl.program_id(0)

    @pl.when(d == 0)
    def _():
        o_ref[...] = x_ref[...]

    x = o_ref[...]
    M, C = x.shape
    q = C // 4

    ridx = lax.broadcasted_iota(jnp.int32, (M, 1), 0)
    h_i = ridx // W
    w_i = ridx - h_i * W
    cq = lax.broadcasted_iota(jnp.int32, (1, C), 1) // q

    def sh_hf(t):
        return jnp.where(h_i == 0, t, pltpu.roll(t, W, 0))

    def sh_hb(t):
        return jnp.where(h_i == H - 1, t, pltpu.roll(t, M - W, 0))

    def sh_wf(t):
        return jnp.where(w_i == 0, t, pltpu.roll(t, 1, 0))

    def sh_wb(t):
        return jnp.where(w_i == W - 1, t, pltpu.roll(t, M - 1, 0))

    def _r(t):
        return t.astype(jnp.bfloat16).astype(jnp.float32)

    ones_row = jnp.full((1, M), 1.0, dtype=jnp.float32)

    for k in range(lps):
        xn = _layernorm(x, ln1g[k], ln1b[k])
        t1 = _dot(xn, w1a[k]) + b1a[k]
        t2 = _dot(xn, w1b[k]) + b1b[k]
        t3 = _dot(xn, w1c[k]) + b1c[k]

        x1s = _r(jnp.where(cq == 0, sh_hf(t1),
                 jnp.where(cq == 1, sh_hb(t1),
                 jnp.where(cq == 2, sh_wf(t1), sh_wb(t1)))))
        x2s = _r(jnp.where(cq == 0, sh_wf(t2),
                 jnp.where(cq == 1, sh_wb(t2),
                 jnp.where(cq == 2, sh_hf(t2), sh_hb(t2)))))

        a = _dot(ones_row, x1s + x2s + t3)
        hmid = _gelu(_dot(a, sa1[k]))
        ha1 = _dot(hmid, sa2a[k])
        ha2 = _dot(hmid, sa2b[k])
        ha3 = _dot(hmid, sa2c[k])
        hmax = jnp.maximum(ha1, jnp.maximum(ha2, ha3))
        e1 = jnp.exp(ha1 - hmax)
        e2 = jnp.exp(ha2 - hmax)
        e3 = jnp.exp(ha3 - hmax)
        inv = 1.0 / (e1 + e2 + e3)
        att = (_r(e1 * inv) * x1s + _r(e2 * inv) * x2s
               + _r(e3 * inv) * t3)
        x = x + (_dot(att, w2[k]) + b2[k])

        xn2 = _layernorm(x, ln2g[k], ln2b[k])
        hh = _gelu(_dot(xn2, f1w[k]) + f1b[k])
        x = x + (_dot(hh, f2w[k]) + f2b[k])

    o_ref[...] = x


_S2_KEYS = ('ln1_g', 'ln1_b', 'w1a', 'w1b', 'w1c', 'b1a', 'b1b', 'b1c',
            'sa1', 'sa2a', 'sa2b', 'sa2c', 'w2', 'b2', 'ln2_g', 'ln2_b',
            'f1w', 'f1b', 'f2w', 'f2b')


def _s2_stage(x, wts):
    B, H, W, C = x.shape
    M = B * H * W
    depth = wts['ln1_g'].shape[0]
    args = [wts[k] for k in _S2_KEYS]

    layer_bytes = sum(int(a.nbytes) for a in args) // depth
    if layer_bytes * depth <= 2 * 2 ** 20:
        lps = depth
    else:
        lps = 1
        for cand_lps in range(depth - 1, 0, -1):
            if depth % cand_lps == 0 and cand_lps * layer_bytes <= 13 * 2 ** 20:
                lps = cand_lps
                break
    steps = depth // lps

    def _pl_spec(a):
        return pl.BlockSpec((lps,) + a.shape[1:], lambda d: (d, 0, 0))

    in_specs = [pl.BlockSpec((M, C), lambda d: (0, 0))]
    in_specs += [_pl_spec(a) for a in args]

    out = pl.pallas_call(
        functools.partial(_s2_stage_body, H=H, W=W, lps=lps),
        out_shape=jax.ShapeDtypeStruct((M, C), jnp.float32),
        grid=(steps,),
        in_specs=in_specs,
        out_specs=pl.BlockSpec((M, C), lambda d: (0, 0)),
        compiler_params=pltpu.CompilerParams(
            dimension_semantics=("arbitrary",)),
    )(x.reshape(M, C), *args)
    return out.reshape(B, H, W, C)


def _row_call(body, M, N, row_args, full_args):
    tm = min(M, 2048)
    in_specs = [pl.BlockSpec((tm, a.shape[-1]), lambda i: (i, 0))
                for a in row_args]
    in_specs += [pl.BlockSpec(a.shape, lambda i: (0, 0)) for a in full_args]
    return pl.pallas_call(
        body,
        out_shape=jax.ShapeDtypeStruct((M, N), jnp.float32),
        grid=(M // tm,),
        in_specs=in_specs,
        out_specs=pl.BlockSpec((tm, N), lambda i: (i, 0)),
        compiler_params=pltpu.CompilerParams(
            dimension_semantics=("parallel",)),
    )(*row_args, *full_args)


def _k_lin_bias(x, w, b, o):
    o[...] = _dot(x[...], w[...]) + b[...]


def _k_ln_lin(x, g, b, w, o):
    o[...] = _dot(_layernorm(x[...], g[...], b[...]), w[...])


def _k_merge_proj(x, g, b, w1, w2, bb, o):
    y = _dot(_layernorm(x[...], g[...], b[...]), w1[...])
    o[...] = _dot(y, w2[...]) + bb[...]


def _merge_rows(xe, xo, C):
    E = xe[...][:, 0]
    O = xo[...][:, 0]
    xc = jnp.concatenate([E[..., :C], O[..., :C], E[..., C:], O[..., C:]],
                         axis=-1)
    s = xc.shape
    return xc.reshape(s[0] * s[1], s[2])


def _k_merge_lin(xe, xo, g, b, w, o, *, C):
    xc = _merge_rows(xe, xo, C)
    o[...] = _dot(_layernorm(xc, g[...], b[...]), w[...])


def _k_merge_lin_proj(xe, xo, g, b, w, w2, bb, o, *, C):
    xc = _merge_rows(xe, xo, C)
    y = _dot(_layernorm(xc, g[...], b[...]), w[...])
    o[...] = _dot(y, w2[...]) + bb[...]


def _merge_call(x, g, b, w, proj=None):
    B, H, W, C = x.shape
    H2, W2 = H // 2, W // 2
    N = (proj[0] if proj else w).shape[1]
    hb = H2
    while hb * W2 > 2048:
        hb //= 2
    xp4 = x.reshape(H2, 2, W2, 2 * C)
    g2 = g.reshape(1, 4 * C)
    b2 = b.reshape(1, 4 * C)
    row_specs = [
        pl.BlockSpec((hb, 1, W2, 2 * C), lambda i: (i, 0, 0, 0)),
        pl.BlockSpec((hb, 1, W2, 2 * C), lambda i: (i, 1, 0, 0)),
    ]
    if proj is None:
        body = functools.partial(_k_merge_lin, C=C)
        full = [g2, b2, w]
    else:
        w2_, b2_ = proj
        body = functools.partial(_k_merge_lin_proj, C=C)
        full = [g2, b2, w, w2_, b2_.reshape(1, N)]
    out = pl.pallas_call(
        body,
        out_shape=jax.ShapeDtypeStruct((H2 * W2, N), jnp.float32),
        grid=(H2 // hb,),
        in_specs=row_specs + [pl.BlockSpec(a.shape, lambda i: (0, 0))
                              for a in full],
        out_specs=pl.BlockSpec((hb * W2, N), lambda i: (i, 0)),
        compiler_params=pltpu.CompilerParams(
            dimension_semantics=("parallel",)),
    )(xp4, xp4, *full)
    return out.reshape(B, H2, W2, N)


def _k_dual_bias(x, y, wx, wy, b, o):
    o[...] = _dot(x[...], wx[...]) + _dot(y[...], wy[...]) + b[...]


def _grouped_ln(t, G, GT, gam, bet, gs):
    mu = _dot(_dot(t, G), GT) / gs
    dd = t - mu
    var = _dot(_dot(dd * dd, G), GT) / gs
    return dd * lax.rsqrt(var + _EPS) * gam + bet


def _k_expand(x, w, G, GT, gam, bet, o, *, gs):
    t = _dot(x[...], w[...])
    o[...] = _grouped_ln(t, G[...], GT[...], gam[...], bet[...], gs)


def _k_ln_expand(x, g0, b0, w, G, GT, gam, bet, o, *, gs):
    t = _dot(_layernorm(x[...], g0[...], b0[...]), w[...])
    o[...] = _grouped_ln(t, G[...], GT[...], gam[...], bet[...], gs)


def _linear_bias(x, w, b):
    lead, K = x.shape[:-1], x.shape[-1]
    N = w.shape[1]
    y = _row_call(_k_lin_bias, x.size // K, N,
                  [x.reshape(-1, K)], [w, b.reshape(1, N)])
    return y.reshape(lead + (N,))


def _space_to_depth2(x):
    B, H, W, C = x.shape
    y = x.reshape(B, H // 2, 2, W // 2, 2, C)
    y = jnp.transpose(y, (0, 1, 3, 4, 2, 5))
    return y.reshape(B, H // 2, W // 2, 4 * C)


def _patch_merge(x, g, b, w):
    if x.shape[2] >= 16:
        return _merge_call(x, g, b, w)
    xc = _space_to_depth2(x)
    lead, K = xc.shape[:-1], xc.shape[-1]
    N = w.shape[1]
    y = _row_call(_k_ln_lin, xc.size // K, N,
                  [xc.reshape(-1, K)], [g.reshape(1, K), b.reshape(1, K), w])
    return y.reshape(lead + (N,))


def _dual_linear(x, y, wx, wy, b):
    lead, K = x.shape[:-1], x.shape[-1]
    N = wx.shape[1]
    out = _row_call(_k_dual_bias, x.size // K, N,
                    [x.reshape(-1, K), y.reshape(-1, y.shape[-1])],
                    [wx, wy, b.reshape(1, N)])
    return out.reshape(lead + (N,))


def _expand_call(x, w, ln_g, ln_b, groups, pre_ln):
    lead, K = x.shape[:-1], x.shape[-1]
    N = w.shape[1]
    gs = N // groups
    G = (jnp.arange(N, dtype=jnp.int32)[:, None] // gs ==
         jnp.arange(groups, dtype=jnp.int32)[None, :]).astype(jnp.float32)
    gam = jnp.tile(ln_g, groups).reshape(1, N)
    bet = jnp.tile(ln_b, groups).reshape(1, N)
    M = x.size // K
    if pre_ln is None:
        y = _row_call(functools.partial(_k_expand, gs=gs), M, N,
                      [x.reshape(-1, K)], [w, G, G.T, gam, bet])
    else:
        g0, b0 = pre_ln
        y = _row_call(functools.partial(_k_ln_expand, gs=gs), M, N,
                      [x.reshape(-1, K)],
                      [g0.reshape(1, K), b0.reshape(1, K), w, G, G.T, gam, bet])
    return y.reshape(lead + (N,))


def _patch_expand(x, w, ln_g, ln_b, pre_ln=None):
    B, H, W, C = x.shape
    y = _expand_call(x, w, ln_g, ln_b, 4, pre_ln)
    y = y.reshape(B, H, W, 2, 2, C // 2)
    return jnp.transpose(y, (0, 1, 3, 2, 4, 5)).reshape(B, 2 * H, 2 * W, C // 2)


def _k_expand4_shuf(x, g0, b0, w, G, GT, gam, bet, o, *, gs, W, hb):
    xn = _layernorm(x[...], g0[...], b0[...])
    t = _grouped_ln(_dot(xn, w[...]), G[...], GT[...], gam[...], bet[...], gs)
    qc = t.shape[1] // 4
    for hp in range(hb):
        for dh in range(4):
            o[(4 * hp + dh) * W:(4 * hp + dh + 1) * W, :] = \
                t[hp * W:(hp + 1) * W, dh * qc:(dh + 1) * qc]


def _final_expand_x4(x, w, ln_g, ln_b, pre_ln):
    B, H, W, C = x.shape
    N = w.shape[1]
    gs = N // 16
    G = (jnp.arange(N, dtype=jnp.int32)[:, None] // gs ==
         jnp.arange(16, dtype=jnp.int32)[None, :]).astype(jnp.float32)
    gam = jnp.tile(ln_g, 16).reshape(1, N)
    bet = jnp.tile(ln_b, 16).reshape(1, N)
    g0, b0 = pre_ln
    hb = 4
    out = pl.pallas_call(
        functools.partial(_k_expand4_shuf, gs=gs, W=W, hb=hb),
        out_shape=jax.ShapeDtypeStruct((4 * H * W, N // 4), jnp.float32),
        grid=(H // hb,),
        in_specs=[pl.BlockSpec((hb * W, C), lambda i: (i, 0))] +
                 [pl.BlockSpec(a.shape, lambda i: (0, 0))
                  for a in (g0.reshape(1, C), b0.reshape(1, C), w, G, G.T,
                            gam, bet)],
        out_specs=pl.BlockSpec((4 * hb * W, N // 4), lambda i: (i, 0)),
        compiler_params=pltpu.CompilerParams(
            dimension_semantics=("parallel",)),
    )(x.reshape(H * W, C), g0.reshape(1, C), b0.reshape(1, C), w, G, G.T,
      gam, bet)
    return out.reshape(B, 4 * H, 4 * W, C)


def kernel(x, embed_w, embed_b, proj_w, proj_b, norm_g, norm_b, norm_up_g, norm_up_b, cls_wa, cls_wb, cls_b, pm0_norm_g, pm0_norm_b, pm0_red_w, pm1_norm_g, pm1_norm_b, pm1_red_w, d1pm_norm_g, d1pm_norm_b, d1pm_red_w, d2pm_norm_g, d2pm_norm_b, d2pm_red_w, d3pm_norm_g, d3pm_norm_b, d3pm_red_w, u3pe_exp_w, u3pe_norm_g, u3pe_norm_b, u2pe_exp_w, u2pe_norm_g, u2pe_norm_b, u1pe_exp_w, u1pe_norm_g, u1pe_norm_b, oc3_wx, oc3_wy, oc3_b, oc2_wx, oc2_wy, oc2_b, oc1_wx, oc1_wy, oc1_b, up4x_exp_w, up4x_norm_g, up4x_norm_b, block_ln1_g, block_ln1_b, block_w1a, block_w1b, block_w1c, block_b1a, block_b1b, block_b1c, block_sa1, block_sa2a, block_sa2b, block_sa2c, block_w2, block_b2, block_ln2_g, block_ln2_b, block_f1w, block_f1b, block_f2w, block_f2b, d1b_ln1_g, d1b_ln1_b, d1b_w1a, d1b_w1b, d1b_w1c, d1b_b1a, d1b_b1b, d1b_b1c, d1b_sa1, d1b_sa2a, d1b_sa2b, d1b_sa2c, d1b_w2, d1b_b2, d1b_ln2_g, d1b_ln2_b, d1b_f1w, d1b_f1b, d1b_f2w, d1b_f2b, d2b_ln1_g, d2b_ln1_b, d2b_w1a, d2b_w1b, d2b_w1c, d2b_b1a, d2b_b1b, d2b_b1c, d2b_sa1, d2b_sa2a, d2b_sa2b, d2b_sa2c, d2b_w2, d2b_b2, d2b_ln2_g, d2b_ln2_b, d2b_f1w, d2b_f1b, d2b_f2w, d2b_f2b, d3b_ln1_g, d3b_ln1_b, d3b_w1a, d3b_w1b, d3b_w1c, d3b_b1a, d3b_b1b, d3b_b1c, d3b_sa1, d3b_sa2a, d3b_sa2b, d3b_sa2c, d3b_w2, d3b_b2, d3b_ln2_g, d3b_ln2_b, d3b_f1w, d3b_f1b, d3b_f2w, d3b_f2b, u3b_ln1_g, u3b_ln1_b, u3b_w1a, u3b_w1b, u3b_w1c, u3b_b1a, u3b_b1b, u3b_b1c, u3b_sa1, u3b_sa2a, u3b_sa2b, u3b_sa2c, u3b_w2, u3b_b2, u3b_ln2_g, u3b_ln2_b, u3b_f1w, u3b_f1b, u3b_f2w, u3b_f2b, u2b_ln1_g, u2b_ln1_b, u2b_w1a, u2b_w1b, u2b_w1c, u2b_b1a, u2b_b1b, u2b_b1c, u2b_sa1, u2b_sa2a, u2b_sa2b, u2b_sa2c, u2b_w2, u2b_b2, u2b_ln2_g, u2b_ln2_b, u2b_f1w, u2b_f1b, u2b_f2w, u2b_f2b, u1b_ln1_g, u1b_ln1_b, u1b_w1a, u1b_w1b, u1b_w1c, u1b_b1a, u1b_b1b, u1b_b1c, u1b_sa1, u1b_sa2a, u1b_sa2b, u1b_sa2c, u1b_w2, u1b_b2, u1b_ln2_g, u1b_ln2_b, u1b_f1w, u1b_f1b, u1b_f2w, u1b_f2b):
    scope = locals()

    def blk(prefix):
        return {k: scope[prefix + '_' + k] for k in _S2_KEYS}

    xh = jnp.transpose(x, (0, 2, 3, 1)).astype(jnp.float32)
    x0 = _linear_bias(xh, embed_w, embed_b)

    t = _patch_merge(x0, pm0_norm_g, pm0_norm_b, pm0_red_w)
    t = _merge_call(t, pm1_norm_g, pm1_norm_b, pm1_red_w,
                    proj=(proj_w, proj_b))

    x1 = _s2_stage(t, blk('block'))
    x2 = _s2_stage(_patch_merge(x1, d1pm_norm_g, d1pm_norm_b, d1pm_red_w),
                   blk('d1b'))
    x3 = _s2_stage(_patch_merge(x2, d2pm_norm_g, d2pm_norm_b, d2pm_red_w),
                   blk('d2b'))
    center = _s2_stage(_patch_merge(x3, d3pm_norm_g, d3pm_norm_b, d3pm_red_w),
                       blk('d3b'))

    u3 = _s2_stage(_patch_expand(center, u3pe_exp_w, u3pe_norm_g, u3pe_norm_b,
                                 pre_ln=(norm_g, norm_b)), blk('u3b'))
    out = _dual_linear(u3, x3, oc3_wx, oc3_wy, oc3_b)
    u2 = _s2_stage(_patch_expand(out, u2pe_exp_w, u2pe_norm_g, u2pe_norm_b),
                   blk('u2b'))
    out = _dual_linear(u2, x2, oc2_wx, oc2_wy, oc2_b)
    u1 = _s2_stage(_patch_expand(out, u1pe_exp_w, u1pe_norm_g, u1pe_norm_b),
                   blk('u1b'))
    out = _dual_linear(u1, x1, oc1_wx, oc1_wy, oc1_b)

    out = _final_expand_x4(out, up4x_exp_w, up4x_norm_g, up4x_norm_b,
                           pre_ln=(norm_up_g, norm_up_b))
    logits = _dual_linear(out, x0, cls_wa, cls_wb, cls_b)
    return jnp.transpose(logits, (0, 3, 1, 2))


# speedup vs baseline: 1.0005x; 1.0005x over previous
import functools

import jax
import jax.numpy as jnp
from jax import lax
from jax.experimental import pallas as pl
from jax.experimental.pallas import tpu as pltpu

_EPS = 1e-5
_INV_SQRT2 = 0.7071067811865476


def _layernorm(v, g, b):
    mu = jnp.mean(v, axis=-1, keepdims=True)
    c = v - mu
    var = jnp.mean(c * c, axis=-1, keepdims=True)
    return c * lax.rsqrt(var + _EPS) * g + b


def _erf_poly(z):
    c1, c2, c3, c4, c5 = (0.254829592, -0.284496736, 1.421413741,
                          -1.453152027, 1.061405429)
    az = jnp.abs(z)
    t = 1.0 / (1.0 + 0.3275911 * az)
    poly = ((((c5 * t + c4) * t + c3) * t + c2) * t + c1) * t
    y = 1.0 - poly * jnp.exp(-az * az)
    return jnp.where(z < 0.0, -y, y)


def _gelu(v):
    return 0.5 * v * (1.0 + _erf_poly(v * _INV_SQRT2))


def _dot(a, b):
    return jnp.dot(a, b, preferred_element_type=jnp.float32)


def _s2_stage_body(x_ref, ln1g, ln1b, w1a, w1b, w1c, b1a, b1b, b1c,
                   sa1, sa2a, sa2b, sa2c, w2, b2, ln2g, ln2b,
                   f1w, f1b, f2w, f2b, o_ref, *, H, W, lps):
    d = pl.program_id(0)

    @pl.when(d == 0)
    def _():
        o_ref[...] = x_ref[...]

    x = o_ref[...]
    M, C = x.shape
    q = C // 4

    ridx = lax.broadcasted_iota(jnp.int32, (M, 1), 0)
    h_i = ridx // W
    w_i = ridx - h_i * W
    cq = lax.broadcasted_iota(jnp.int32, (1, C), 1) // q

    def sh_hf(t):
        return jnp.where(h_i == 0, t, pltpu.roll(t, W, 0))

    def sh_hb(t):
        return jnp.where(h_i == H - 1, t, pltpu.roll(t, M - W, 0))

    def sh_wf(t):
        return jnp.where(w_i == 0, t, pltpu.roll(t, 1, 0))

    def sh_wb(t):
        return jnp.where(w_i == W - 1, t, pltpu.roll(t, M - 1, 0))

    def _r(t):
        return t.astype(jnp.bfloat16).astype(jnp.float32)

    ones_row = jnp.full((1, M), 1.0, dtype=jnp.float32)

    for k in range(lps):
        xn = _layernorm(x, ln1g[k], ln1b[k])
        t1 = _dot(xn, w1a[k]) + b1a[k]
        t2 = _dot(xn, w1b[k]) + b1b[k]
        t3 = _dot(xn, w1c[k]) + b1c[k]

        x1s = _r(jnp.where(cq == 0, sh_hf(t1),
                 jnp.where(cq == 1, sh_hb(t1),
                 jnp.where(cq == 2, sh_wf(t1), sh_wb(t1)))))
        x2s = _r(jnp.where(cq == 0, sh_wf(t2),
                 jnp.where(cq == 1, sh_wb(t2),
                 jnp.where(cq == 2, sh_hf(t2), sh_hb(t2)))))

        a = _dot(ones_row, x1s + x2s + t3)
        hmid = _gelu(_dot(a, sa1[k]))
        ha1 = _dot(hmid, sa2a[k])
        ha2 = _dot(hmid, sa2b[k])
        ha3 = _dot(hmid, sa2c[k])
        hmax = jnp.maximum(ha1, jnp.maximum(ha2, ha3))
        e1 = jnp.exp(ha1 - hmax)
        e2 = jnp.exp(ha2 - hmax)
        e3 = jnp.exp(ha3 - hmax)
        inv = 1.0 / (e1 + e2 + e3)
        att = (_r(e1 * inv) * x1s + _r(e2 * inv) * x2s
               + _r(e3 * inv) * t3)
        x = x + (_dot(att, w2[k]) + b2[k])

        xn2 = _layernorm(x, ln2g[k], ln2b[k])
        hh = _gelu(_dot(xn2, f1w[k]) + f1b[k])
        x = x + (_dot(hh, f2w[k]) + f2b[k])

    o_ref[...] = x


_S2_KEYS = ('ln1_g', 'ln1_b', 'w1a', 'w1b', 'w1c', 'b1a', 'b1b', 'b1c',
            'sa1', 'sa2a', 'sa2b', 'sa2c', 'w2', 'b2', 'ln2_g', 'ln2_b',
            'f1w', 'f1b', 'f2w', 'f2b')


def _s2_stage(x, wts):
    B, H, W, C = x.shape
    M = B * H * W
    depth = wts['ln1_g'].shape[0]
    args = [wts[k] for k in _S2_KEYS]

    layer_bytes = sum(int(a.nbytes) for a in args) // depth
    if layer_bytes * depth <= 2 * 2 ** 20:
        lps = depth
    else:
        lps = 1
        for cand_lps in range(depth - 1, 0, -1):
            if depth % cand_lps == 0 and cand_lps * layer_bytes <= 13 * 2 ** 20:
                lps = cand_lps
                break
    steps = depth // lps

    def _pl_spec(a):
        return pl.BlockSpec((lps,) + a.shape[1:], lambda d: (d, 0, 0))

    in_specs = [pl.BlockSpec((M, C), lambda d: (0, 0))]
    in_specs += [_pl_spec(a) for a in args]

    out = pl.pallas_call(
        functools.partial(_s2_stage_body, H=H, W=W, lps=lps),
        out_shape=jax.ShapeDtypeStruct((M, C), jnp.float32),
        grid=(steps,),
        in_specs=in_specs,
        out_specs=pl.BlockSpec((M, C), lambda d: (0, 0)),
        compiler_params=pltpu.CompilerParams(
            dimension_semantics=("arbitrary",)),
    )(x.reshape(M, C), *args)
    return out.reshape(B, H, W, C)


def _row_call(body, M, N, row_args, full_args):
    tm = min(M, 2048)
    in_specs = [pl.BlockSpec((tm, a.shape[-1]), lambda i: (i, 0))
                for a in row_args]
    in_specs += [pl.BlockSpec(a.shape, lambda i: (0, 0)) for a in full_args]
    return pl.pallas_call(
        body,
        out_shape=jax.ShapeDtypeStruct((M, N), jnp.float32),
        grid=(M // tm,),
        in_specs=in_specs,
        out_specs=pl.BlockSpec((tm, N), lambda i: (i, 0)),
        compiler_params=pltpu.CompilerParams(
            dimension_semantics=("parallel",)),
    )(*row_args, *full_args)


def _k_lin_bias(x, w, b, o):
    o[...] = _dot(x[...], w[...]) + b[...]


def _k_ln_lin(x, g, b, w, o):
    o[...] = _dot(_layernorm(x[...], g[...], b[...]), w[...])


def _merge_rows(xe, xo, C):
    E = xe[...][:, 0]
    O = xo[...][:, 0]
    xc = jnp.concatenate([E[..., :C], O[..., :C], E[..., C:], O[..., C:]],
                         axis=-1)
    s = xc.shape
    return xc.reshape(s[0] * s[1], s[2])


def _k_merge_lin(xe, xo, g, b, w, o, *, C):
    xc = _merge_rows(xe, xo, C)
    o[...] = _dot(_layernorm(xc, g[...], b[...]), w[...])


def _k_merge_lin_proj(xe, xo, g, b, w, w2, bb, o, *, C):
    xc = _merge_rows(xe, xo, C)
    y = _dot(_layernorm(xc, g[...], b[...]), w[...])
    o[...] = _dot(y, w2[...]) + bb[...]


def _merge_call(x, g, b, w, proj=None):
    B, H, W, C = x.shape
    H2, W2 = H // 2, W // 2
    N = (proj[0] if proj else w).shape[1]
    hb = H2
    while hb * W2 > 2048:
        hb //= 2
    xp4 = x.reshape(H2, 2, W2, 2 * C)
    g2 = g.reshape(1, 4 * C)
    b2 = b.reshape(1, 4 * C)
    row_specs = [
        pl.BlockSpec((hb, 1, W2, 2 * C), lambda i: (i, 0, 0, 0)),
        pl.BlockSpec((hb, 1, W2, 2 * C), lambda i: (i, 1, 0, 0)),
    ]
    if proj is None:
        body = functools.partial(_k_merge_lin, C=C)
        full = [g2, b2, w]
    else:
        w2_, b2_ = proj
        body = functools.partial(_k_merge_lin_proj, C=C)
        full = [g2, b2, w, w2_, b2_.reshape(1, N)]
    out = pl.pallas_call(
        body,
        out_shape=jax.ShapeDtypeStruct((H2 * W2, N), jnp.float32),
        grid=(H2 // hb,),
        in_specs=row_specs + [pl.BlockSpec(a.shape, lambda i: (0, 0))
                              for a in full],
        out_specs=pl.BlockSpec((hb * W2, N), lambda i: (i, 0)),
        compiler_params=pltpu.CompilerParams(
            dimension_semantics=("parallel",)),
    )(xp4, xp4, *full)
    return out.reshape(B, H2, W2, N)


def _k_dual_bias(x, y, wx, wy, b, o):
    o[...] = _dot(x[...], wx[...]) + _dot(y[...], wy[...]) + b[...]


def _grouped_ln(t, G, GT, gam, bet, gs):
    mu = _dot(_dot(t, G), GT) / gs
    dd = t - mu
    var = _dot(_dot(dd * dd, G), GT) / gs
    return dd * lax.rsqrt(var + _EPS) * gam + bet


def _k_expand(x, w, G, GT, gam, bet, o, *, gs):
    t = _dot(x[...], w[...])
    o[...] = _grouped_ln(t, G[...], GT[...], gam[...], bet[...], gs)


def _k_ln_expand(x, g0, b0, w, G, GT, gam, bet, o, *, gs):
    t = _dot(_layernorm(x[...], g0[...], b0[...]), w[...])
    o[...] = _grouped_ln(t, G[...], GT[...], gam[...], bet[...], gs)


def _linear_bias(x, w, b):
    lead, K = x.shape[:-1], x.shape[-1]
    N = w.shape[1]
    y = _row_call(_k_lin_bias, x.size // K, N,
                  [x.reshape(-1, K)], [w, b.reshape(1, N)])
    return y.reshape(lead + (N,))


def _space_to_depth2(x):
    B, H, W, C = x.shape
    y = x.reshape(B, H // 2, 2, W // 2, 2, C)
    y = jnp.transpose(y, (0, 1, 3, 4, 2, 5))
    return y.reshape(B, H // 2, W // 2, 4 * C)


def _patch_merge(x, g, b, w):
    if x.shape[2] >= 16:
        return _merge_call(x, g, b, w)
    xc = _space_to_depth2(x)
    lead, K = xc.shape[:-1], xc.shape[-1]
    N = w.shape[1]
    y = _row_call(_k_ln_lin, xc.size // K, N,
                  [xc.reshape(-1, K)], [g.reshape(1, K), b.reshape(1, K), w])
    return y.reshape(lead + (N,))


def _dual_linear(x, y, wx, wy, b):
    lead, K = x.shape[:-1], x.shape[-1]
    N = wx.shape[1]
    out = _row_call(_k_dual_bias, x.size // K, N,
                    [x.reshape(-1, K), y.reshape(-1, y.shape[-1])],
                    [wx, wy, b.reshape(1, N)])
    return out.reshape(lead + (N,))


def _expand_call(x, w, ln_g, ln_b, groups, pre_ln):
    lead, K = x.shape[:-1], x.shape[-1]
    N = w.shape[1]
    gs = N // groups
    G = (jnp.arange(N, dtype=jnp.int32)[:, None] // gs ==
         jnp.arange(groups, dtype=jnp.int32)[None, :]).astype(jnp.float32)
    gam = jnp.tile(ln_g, groups).reshape(1, N)
    bet = jnp.tile(ln_b, groups).reshape(1, N)
    M = x.size // K
    if pre_ln is None:
        y = _row_call(functools.partial(_k_expand, gs=gs), M, N,
                      [x.reshape(-1, K)], [w, G, G.T, gam, bet])
    else:
        g0, b0 = pre_ln
        y = _row_call(functools.partial(_k_ln_expand, gs=gs), M, N,
                      [x.reshape(-1, K)],
                      [g0.reshape(1, K), b0.reshape(1, K), w, G, G.T, gam, bet])
    return y.reshape(lead + (N,))


def _patch_expand(x, w, ln_g, ln_b, pre_ln=None):
    B, H, W, C = x.shape
    y = _expand_call(x, w, ln_g, ln_b, 4, pre_ln)
    y = y.reshape(B, H, W, 2, 2, C // 2)
    return jnp.transpose(y, (0, 1, 3, 2, 4, 5)).reshape(B, 2 * H, 2 * W, C // 2)


def _k_expand4_shuf(x, g0, b0, w, G, GT, gam, bet, o, *, gs, W, hb):
    xn = _layernorm(x[...], g0[...], b0[...])
    t = _grouped_ln(_dot(xn, w[...]), G[...], GT[...], gam[...], bet[...], gs)
    qc = t.shape[1] // 4
    for hp in range(hb):
        for dh in range(4):
            o[(4 * hp + dh) * W:(4 * hp + dh + 1) * W, :] = \
                t[hp * W:(hp + 1) * W, dh * qc:(dh + 1) * qc]


def _final_expand_x4(x, w, ln_g, ln_b, pre_ln):
    B, H, W, C = x.shape
    N = w.shape[1]
    gs = N // 16
    G = (jnp.arange(N, dtype=jnp.int32)[:, None] // gs ==
         jnp.arange(16, dtype=jnp.int32)[None, :]).astype(jnp.float32)
    gam = jnp.tile(ln_g, 16).reshape(1, N)
    bet = jnp.tile(ln_b, 16).reshape(1, N)
    g0, b0 = pre_ln
    hb = 4
    out = pl.pallas_call(
        functools.partial(_k_expand4_shuf, gs=gs, W=W, hb=hb),
        out_shape=jax.ShapeDtypeStruct((4 * H * W, N // 4), jnp.float32),
        grid=(H // hb,),
        in_specs=[pl.BlockSpec((hb * W, C), lambda i: (i, 0))] +
                 [pl.BlockSpec(a.shape, lambda i: (0, 0))
                  for a in (g0.reshape(1, C), b0.reshape(1, C), w, G, G.T,
                            gam, bet)],
        out_specs=pl.BlockSpec((4 * hb * W, N // 4), lambda i: (i, 0)),
        compiler_params=pltpu.CompilerParams(
            dimension_semantics=("parallel",)),
    )(x.reshape(H * W, C), g0.reshape(1, C), b0.reshape(1, C), w, G, G.T,
      gam, bet)
    return out.reshape(B, 4 * H, 4 * W, C)


def kernel(x, embed_w, embed_b, proj_w, proj_b, norm_g, norm_b, norm_up_g, norm_up_b, cls_wa, cls_wb, cls_b, pm0_norm_g, pm0_norm_b, pm0_red_w, pm1_norm_g, pm1_norm_b, pm1_red_w, d1pm_norm_g, d1pm_norm_b, d1pm_red_w, d2pm_norm_g, d2pm_norm_b, d2pm_red_w, d3pm_norm_g, d3pm_norm_b, d3pm_red_w, u3pe_exp_w, u3pe_norm_g, u3pe_norm_b, u2pe_exp_w, u2pe_norm_g, u2pe_norm_b, u1pe_exp_w, u1pe_norm_g, u1pe_norm_b, oc3_wx, oc3_wy, oc3_b, oc2_wx, oc2_wy, oc2_b, oc1_wx, oc1_wy, oc1_b, up4x_exp_w, up4x_norm_g, up4x_norm_b, block_ln1_g, block_ln1_b, block_w1a, block_w1b, block_w1c, block_b1a, block_b1b, block_b1c, block_sa1, block_sa2a, block_sa2b, block_sa2c, block_w2, block_b2, block_ln2_g, block_ln2_b, block_f1w, block_f1b, block_f2w, block_f2b, d1b_ln1_g, d1b_ln1_b, d1b_w1a, d1b_w1b, d1b_w1c, d1b_b1a, d1b_b1b, d1b_b1c, d1b_sa1, d1b_sa2a, d1b_sa2b, d1b_sa2c, d1b_w2, d1b_b2, d1b_ln2_g, d1b_ln2_b, d1b_f1w, d1b_f1b, d1b_f2w, d1b_f2b, d2b_ln1_g, d2b_ln1_b, d2b_w1a, d2b_w1b, d2b_w1c, d2b_b1a, d2b_b1b, d2b_b1c, d2b_sa1, d2b_sa2a, d2b_sa2b, d2b_sa2c, d2b_w2, d2b_b2, d2b_ln2_g, d2b_ln2_b, d2b_f1w, d2b_f1b, d2b_f2w, d2b_f2b, d3b_ln1_g, d3b_ln1_b, d3b_w1a, d3b_w1b, d3b_w1c, d3b_b1a, d3b_b1b, d3b_b1c, d3b_sa1, d3b_sa2a, d3b_sa2b, d3b_sa2c, d3b_w2, d3b_b2, d3b_ln2_g, d3b_ln2_b, d3b_f1w, d3b_f1b, d3b_f2w, d3b_f2b, u3b_ln1_g, u3b_ln1_b, u3b_w1a, u3b_w1b, u3b_w1c, u3b_b1a, u3b_b1b, u3b_b1c, u3b_sa1, u3b_sa2a, u3b_sa2b, u3b_sa2c, u3b_w2, u3b_b2, u3b_ln2_g, u3b_ln2_b, u3b_f1w, u3b_f1b, u3b_f2w, u3b_f2b, u2b_ln1_g, u2b_ln1_b, u2b_w1a, u2b_w1b, u2b_w1c, u2b_b1a, u2b_b1b, u2b_b1c, u2b_sa1, u2b_sa2a, u2b_sa2b, u2b_sa2c, u2b_w2, u2b_b2, u2b_ln2_g, u2b_ln2_b, u2b_f1w, u2b_f1b, u2b_f2w, u2b_f2b, u1b_ln1_g, u1b_ln1_b, u1b_w1a, u1b_w1b, u1b_w1c, u1b_b1a, u1b_b1b, u1b_b1c, u1b_sa1, u1b_sa2a, u1b_sa2b, u1b_sa2c, u1b_w2, u1b_b2, u1b_ln2_g, u1b_ln2_b, u1b_f1w, u1b_f1b, u1b_f2w, u1b_f2b):
    scope = locals()

    def blk(prefix):
        return {k: scope[prefix + '_' + k] for k in _S2_KEYS}

    xh = jnp.transpose(x, (0, 2, 3, 1)).astype(jnp.float32)
    x0 = _linear_bias(xh, embed_w, embed_b)

    t = _patch_merge(x0, pm0_norm_g, pm0_norm_b, pm0_red_w)
    t = _merge_call(t, pm1_norm_g, pm1_norm_b, pm1_red_w,
                    proj=(proj_w, proj_b))

    x1 = _s2_stage(t, blk('block'))
    x2 = _s2_stage(_patch_merge(x1, d1pm_norm_g, d1pm_norm_b, d1pm_red_w),
                   blk('d1b'))
    x3 = _s2_stage(_patch_merge(x2, d2pm_norm_g, d2pm_norm_b, d2pm_red_w),
                   blk('d2b'))
    center = _s2_stage(_patch_merge(x3, d3pm_norm_g, d3pm_norm_b, d3pm_red_w),
                       blk('d3b'))

    u3 = _s2_stage(_patch_expand(center, u3pe_exp_w, u3pe_norm_g, u3pe_norm_b,
                                 pre_ln=(norm_g, norm_b)), blk('u3b'))
    out = _dual_linear(u3, x3, oc3_wx, oc3_wy, oc3_b)
    u2 = _s2_stage(_patch_expand(out, u2pe_exp_w, u2pe_norm_g, u2pe_norm_b),
                   blk('u2b'))
    out = _dual_linear(u2, x2, oc2_wx, oc2_wy, oc2_b)
    u1 = _s2_stage(_patch_expand(out, u1pe_exp_w, u1pe_norm_g, u1pe_norm_b),
                   blk('u1b'))
    out = _dual_linear(u1, x1, oc1_wx, oc1_wy, oc1_b)

    out = _final_expand_x4(out, up4x_exp_w, up4x_norm_g, up4x_norm_b,
                           pre_ln=(norm_up_g, norm_up_b))
    logits = _dual_linear(out, x0, cls_wa, cls_wb, cls_b)
    return jnp.transpose(logits, (0, 3, 1, 2))
